# baseline (speedup 1.0000x reference)
import jax
import jax.numpy as jnp
from jax import lax
from jax.experimental import pallas as pl
from jax.experimental.pallas import tpu as pltpu
from jax.experimental.pallas import tpu_sc as plsc

_NW = 32
_RCS = (64, 64, 64, 64, 64, 64, 64, 40)
_RCMAX = 64
_WR = sum(_RCS)
_N = 2_000_000
_R = _N // 128
_TROW = _NW * _WR
_LROW = _TROW + 8
_TBL = 64


def _sc_body(x_hbm, xl_hbm, z_hbm, t_hbm, out_hbm, tbl_v,
             zb0, zb1, xb0, xb1, ob0, ob1,
             sz0, sz1, sx0, sx1, so0, so1):
    wid = lax.axis_index("s") * 2 + lax.axis_index("c")
    pltpu.sync_copy(t_hbm, tbl_v)
    zb, xb, ob = (zb0, zb1), (xb0, xb1), (ob0, ob1)
    sz, sx, so = (sz0, sz1), (sx0, sx1), (so0, so1)
    row0 = wid * _WR
    starts = [sum(_RCS[:i]) for i in range(len(_RCS))]

    def start_in(c, b):
        r = row0 + starts[c]
        rc = _RCS[c]
        dz = pltpu.async_copy(z_hbm.at[pl.ds(r * 128, rc * 128)],
                              zb[b].at[pl.ds(0, rc * 128)], sz[b])
        dx = pltpu.async_copy(x_hbm.at[pl.ds(r, rc)],
                              xb[b].at[pl.ds(0, rc)], sx[b])
        return dz, dx

    def compute(zv, xv, ov, rows):
        @plsc.parallel_loop(0, rows, 1)
        def _compute(r):
            for l in range(8):
                s = l * 16
                idx = zv[pl.ds(r * 128 + s, 16)]
                sh = plsc.load_gather(tbl_v, [idx])
                ov[pl.ds(r * 128 + s, 16)] = xv[r, pl.ds(s, 16)] + sh

    in_d = {0: start_in(0, 0)}
    out_d = {}
    nchunks = len(_RCS)
    for c in range(nchunks):
        cur = c & 1
        if c + 1 < nchunks:
            in_d[c + 1] = start_in(c + 1, cur ^ 1)
        dz, dx = in_d.pop(c)
        dz.wait()
        dx.wait()
        if c >= 2:
            out_d.pop(c - 2).wait()
        compute(zb[cur], xb[cur], ob[cur], _RCS[c])
        out_d[c] = pltpu.async_copy(
            ob[cur].at[pl.ds(0, _RCS[c] * 128)],
            out_hbm.at[pl.ds((row0 + starts[c]) * 128, _RCS[c] * 128)],
            so[cur])

    for c in sorted(out_d):
        out_d[c].wait()

    @pl.when(wid == 0)
    def _tail():
        pltpu.sync_copy(z_hbm.at[pl.ds(_TROW * 128, 8 * 128)],
                        zb0.at[pl.ds(0, 8 * 128)])
        pltpu.sync_copy(x_hbm.at[pl.ds(_TROW, 8)], xb0.at[pl.ds(0, 8)])
        compute(zb0, xb0, ob0, 8)
        pltpu.sync_copy(ob0.at[pl.ds(0, 8 * 128)],
                        out_hbm.at[pl.ds(_TROW * 128, 8 * 128)])
        pltpu.sync_copy(z_hbm.at[pl.ds(_LROW * 128, 128)],
                        zb0.at[pl.ds(0, 128)])
        pltpu.sync_copy(xl_hbm, ob1.at[pl.ds(0, 128)])

        def body(l, carry):
            s = l * 16
            idx = zb0[pl.ds(s, 16)]
            sh = plsc.load_gather(tbl_v, [idx])
            ob0[pl.ds(s, 16)] = ob1[pl.ds(s, 16)] + sh
            return carry

        lax.fori_loop(0, 8, body, 0)
        pltpu.sync_copy(ob0.at[pl.ds(0, 128)],
                        out_hbm.at[pl.ds(_LROW * 128, 128)])


def kernel(inputs, z, shift_table):
    n = inputs.shape[0]
    x2 = inputs.reshape(_R, 128) + shift_table[0, 0] * 0.0
    xlast = lax.slice(inputs, (_LROW * 128, 0), (n, 1)).reshape(128)
    zi = z.astype(jnp.int32)
    tbl = jnp.zeros((_TBL,), jnp.float32)
    tbl = tbl.at[: shift_table.shape[0]].set(shift_table.reshape(-1))
    mesh = plsc.VectorSubcoreMesh(core_axis_name="c", subcore_axis_name="s")
    out = pl.kernel(
        _sc_body,
        out_type=jax.ShapeDtypeStruct((n,), jnp.float32),
        mesh=mesh,
        compiler_params=pltpu.CompilerParams(needs_layout_passes=False),
        scratch_types=[
            pltpu.VMEM((_TBL,), jnp.float32),
            pltpu.VMEM((_RCMAX * 128,), jnp.int32),
            pltpu.VMEM((_RCMAX * 128,), jnp.int32),
            pltpu.VMEM((_RCMAX, 128), jnp.float32),
            pltpu.VMEM((_RCMAX, 128), jnp.float32),
            pltpu.VMEM((_RCMAX * 128,), jnp.float32),
            pltpu.VMEM((_RCMAX * 128,), jnp.float32),
            pltpu.SemaphoreType.DMA,
            pltpu.SemaphoreType.DMA,
            pltpu.SemaphoreType.DMA,
            pltpu.SemaphoreType.DMA,
            pltpu.SemaphoreType.DMA,
            pltpu.SemaphoreType.DMA,
        ],
    )(x2, xlast, zi, tbl)
    return out.reshape(n, 1)


# speedup vs baseline: 2.2597x; 2.2597x over previous
import jax
import jax.numpy as jnp
from jax import lax
from jax.experimental import pallas as pl
from jax.experimental.pallas import tpu as pltpu
from jax.experimental.pallas import tpu_sc as plsc

_NW = 32
_RC = 61
_CHUNKS = 8
_WR = _RC * _CHUNKS
_N = 2_000_000
_R = _N // 128
_TROW = _NW * _WR
_TAILR = _R - _TROW
_TBL = 64


def _sc_body(x_hbm, z_hbm, t_hbm, out_hbm, tbl_v,
             zb0, zb1, xb0, xb1, ob0, ob1,
             sz0, sz1, sx0, sx1, so0, so1):
    wid = lax.axis_index("s") * 2 + lax.axis_index("c")
    pltpu.sync_copy(t_hbm, tbl_v)
    zb, xb, ob = (zb0, zb1), (xb0, xb1), (ob0, ob1)
    sz, sx, so = (sz0, sz1), (sx0, sx1), (so0, so1)
    row0 = wid * _WR

    def start_in(c, b):
        r = row0 + c * _RC
        dz = pltpu.async_copy(z_hbm.at[pl.ds(r * 128, _RC * 128)],
                              zb[b], sz[b])
        dx = pltpu.async_copy(x_hbm.at[pl.ds(r, _RC)], xb[b], sx[b])
        return dz, dx

    def compute(zv, xv, ov, rows):
        @plsc.parallel_loop(0, rows, 1)
        def _compute(r):
            for l in range(8):
                s = l * 16
                idx = zv[pl.ds(r * 128 + s, 16)]
                sh = plsc.load_gather(tbl_v, [idx])
                ov[pl.ds(r * 128 + s, 16)] = xv[r, 0, pl.ds(s, 16)] + sh

    in_d = {0: start_in(0, 0)}
    out_d = {}
    for c in range(_CHUNKS):
        cur = c & 1
        if c + 1 < _CHUNKS:
            in_d[c + 1] = start_in(c + 1, cur ^ 1)
        dz, dx = in_d.pop(c)
        dz.wait()
        dx.wait()
        if c >= 2:
            out_d.pop(c - 2).wait()
        compute(zb[cur], xb[cur], ob[cur], _RC)
        out_d[c] = pltpu.async_copy(
            ob[cur],
            out_hbm.at[pl.ds((row0 + c * _RC) * 128, _RC * 128)], so[cur])

    for c in sorted(out_d):
        out_d[c].wait()

    @pl.when(wid == 0)
    def _tail():
        pltpu.sync_copy(z_hbm.at[pl.ds(_TROW * 128, _TAILR * 128)],
                        zb0.at[pl.ds(0, _TAILR * 128)])
        pltpu.sync_copy(x_hbm.at[pl.ds(_TROW, _TAILR)],
                        xb0.at[pl.ds(0, _TAILR)])
        compute(zb0, xb0, ob0, _TAILR)
        pltpu.sync_copy(ob0.at[pl.ds(0, _TAILR * 128)],
                        out_hbm.at[pl.ds(_TROW * 128, _TAILR * 128)])


def kernel(inputs, z, shift_table):
    n = inputs.shape[0]
    x3 = inputs.reshape(_R, 1, 128)
    zi = z.astype(jnp.int32)
    tbl = jnp.zeros((_TBL,), jnp.float32)
    tbl = tbl.at[: shift_table.shape[0]].set(shift_table.reshape(-1))
    mesh = plsc.VectorSubcoreMesh(core_axis_name="c", subcore_axis_name="s")
    out = pl.kernel(
        _sc_body,
        out_type=jax.ShapeDtypeStruct((n,), jnp.float32),
        mesh=mesh,
        compiler_params=pltpu.CompilerParams(needs_layout_passes=False),
        scratch_types=[
            pltpu.VMEM((_TBL,), jnp.float32),
            pltpu.VMEM((_RC * 128,), jnp.int32),
            pltpu.VMEM((_RC * 128,), jnp.int32),
            pltpu.VMEM((_RC, 1, 128), jnp.float32),
            pltpu.VMEM((_RC, 1, 128), jnp.float32),
            pltpu.VMEM((_RC * 128,), jnp.float32),
            pltpu.VMEM((_RC * 128,), jnp.float32),
            pltpu.SemaphoreType.DMA,
            pltpu.SemaphoreType.DMA,
            pltpu.SemaphoreType.DMA,
            pltpu.SemaphoreType.DMA,
            pltpu.SemaphoreType.DMA,
            pltpu.SemaphoreType.DMA,
        ],
    )(x3, zi, tbl)
    return out.reshape(n, 1)


# speedup vs baseline: 4.2101x; 1.8631x over previous
import jax
import jax.numpy as jnp
from jax import lax
from jax.experimental import pallas as pl
from jax.experimental.pallas import tpu as pltpu
from jax.experimental.pallas import tpu_sc as plsc

_NW = 32
_RC = 61
_CHUNKS = 8
_WR = _RC * _CHUNKS
_N = 2_000_000
_R = _N // 128
_TROW = _NW * _WR
_TAILR = _R - _TROW
_TBL = 64


def _sc_body(x_hbm, z_hbm, t_hbm, out_hbm, tbl_v,
             zb0, zb1, xb0, xb1, ob0, ob1,
             sz0, sz1, sx0, sx1, so0, so1):
    wid = lax.axis_index("s") * 2 + lax.axis_index("c")
    pltpu.sync_copy(t_hbm, tbl_v)
    zb, xb, ob = (zb0, zb1), (xb0, xb1), (ob0, ob1)
    sz, sx, so = (sz0, sz1), (sx0, sx1), (so0, so1)
    row0 = wid * _WR

    def start_in(c, b):
        r = row0 + c * _RC
        dz = pltpu.async_copy(z_hbm.at[pl.ds(r * 128, _RC * 128)],
                              zb[b], sz[b])
        dx = pltpu.async_copy(x_hbm.at[pl.ds(r, _RC)], xb[b], sx[b])
        return dz, dx

    def compute(zv, xv, ov, rows):
        @plsc.parallel_loop(0, rows, 1)
        def _compute(r):
            for l in range(8):
                s = l * 16
                idx = zv[pl.ds(r * 128 + s, 16)]
                sh = plsc.load_gather(tbl_v, [idx])
                ov[r, 0, pl.ds(s, 16)] = xv[r, 0, pl.ds(s, 16)] + sh

    in_d = {0: start_in(0, 0)}
    out_d = {}
    for c in range(_CHUNKS):
        cur = c & 1
        if c + 1 < _CHUNKS:
            in_d[c + 1] = start_in(c + 1, cur ^ 1)
        dz, dx = in_d.pop(c)
        dz.wait()
        dx.wait()
        if c >= 2:
            out_d.pop(c - 2).wait()
        compute(zb[cur], xb[cur], ob[cur], _RC)
        out_d[c] = pltpu.async_copy(
            ob[cur], out_hbm.at[pl.ds(row0 + c * _RC, _RC)], so[cur])

    for c in sorted(out_d):
        out_d[c].wait()

    @pl.when(wid == 0)
    def _tail():
        pltpu.sync_copy(z_hbm.at[pl.ds(_TROW * 128, _TAILR * 128)],
                        zb0.at[pl.ds(0, _TAILR * 128)])
        pltpu.sync_copy(x_hbm.at[pl.ds(_TROW, _TAILR)],
                        xb0.at[pl.ds(0, _TAILR)])
        compute(zb0, xb0, ob0, _TAILR)
        pltpu.sync_copy(ob0.at[pl.ds(0, _TAILR)],
                        out_hbm.at[pl.ds(_TROW, _TAILR)])


def kernel(inputs, z, shift_table):
    n = inputs.shape[0]
    x3 = inputs.reshape(_R, 1, 128)
    zi = z.astype(jnp.int32)
    tbl = jnp.zeros((_TBL,), jnp.float32)
    tbl = tbl.at[: shift_table.shape[0]].set(shift_table.reshape(-1))
    mesh = plsc.VectorSubcoreMesh(core_axis_name="c", subcore_axis_name="s")
    out = pl.kernel(
        _sc_body,
        out_type=jax.ShapeDtypeStruct((_R, 1, 128), jnp.float32),
        mesh=mesh,
        compiler_params=pltpu.CompilerParams(needs_layout_passes=False),
        scratch_types=[
            pltpu.VMEM((_TBL,), jnp.float32),
            pltpu.VMEM((_RC * 128,), jnp.int32),
            pltpu.VMEM((_RC * 128,), jnp.int32),
            pltpu.VMEM((_RC, 1, 128), jnp.float32),
            pltpu.VMEM((_RC, 1, 128), jnp.float32),
            pltpu.VMEM((_RC, 1, 128), jnp.float32),
            pltpu.VMEM((_RC, 1, 128), jnp.float32),
            pltpu.SemaphoreType.DMA,
            pltpu.SemaphoreType.DMA,
            pltpu.SemaphoreType.DMA,
            pltpu.SemaphoreType.DMA,
            pltpu.SemaphoreType.DMA,
            pltpu.SemaphoreType.DMA,
        ],
    )(x3, zi, tbl)
    return out.reshape(n, 1)


# speedup vs baseline: 4.4419x; 1.0551x over previous
import jax
import jax.numpy as jnp
from jax import lax
from jax.experimental import pallas as pl
from jax.experimental.pallas import tpu as pltpu
from jax.experimental.pallas import tpu_sc as plsc

_NW = 32
_RC = 61
_CHUNKS = 8
_WR = _RC * _CHUNKS
_N = 2_000_000
_R = _N // 128
_TROW = _NW * _WR
_TAILR = _R - _TROW
_TBL = 64


def _sc_body(x_hbm, z_hbm, t_hbm, out_hbm, tbl_v,
             zb0, zb1, xb0, xb1, ob0, ob1, zbt, xbt,
             sz0, sz1, sx0, sx1, so0, so1, st, stz, stx):
    wid = lax.axis_index("s") * 2 + lax.axis_index("c")
    dt = pltpu.async_copy(t_hbm, tbl_v.at[pl.ds(0, 54)], st)
    dtz = pltpu.async_copy(z_hbm.at[pl.ds(_TROW * 128, _TAILR * 128)],
                           zbt, stz)
    dtx = pltpu.async_copy(x_hbm.at[pl.ds(_TROW, _TAILR)], xbt, stx)
    zb, xb, ob = (zb0, zb1), (xb0, xb1), (ob0, ob1)
    sz, sx, so = (sz0, sz1), (sx0, sx1), (so0, so1)
    row0 = wid * _WR

    def start_in(c, b):
        r = row0 + c * _RC
        dz = pltpu.async_copy(z_hbm.at[pl.ds(r * 128, _RC * 128)],
                              zb[b], sz[b])
        dx = pltpu.async_copy(x_hbm.at[pl.ds(r, _RC)], xb[b], sx[b])
        return dz, dx

    def compute(zv, xv, ov, rows):
        @plsc.parallel_loop(0, rows, 1)
        def _compute(r):
            for l in range(8):
                s = l * 16
                idx = zv[pl.ds(r * 128 + s, 16)]
                sh = plsc.load_gather(tbl_v, [idx])
                ov[r, 0, pl.ds(s, 16)] = xv[r, 0, pl.ds(s, 16)] + sh

    in_d = {0: start_in(0, 0)}
    out_d = {}
    dt.wait()
    for c in range(_CHUNKS):
        cur = c & 1
        if c + 1 < _CHUNKS:
            in_d[c + 1] = start_in(c + 1, cur ^ 1)
        dz, dx = in_d.pop(c)
        dz.wait()
        dx.wait()
        if c >= 2:
            out_d.pop(c - 2).wait()
        compute(zb[cur], xb[cur], ob[cur], _RC)
        out_d[c] = pltpu.async_copy(
            ob[cur], out_hbm.at[pl.ds(row0 + c * _RC, _RC)], so[cur])

    for c in sorted(out_d):
        out_d[c].wait()
    dtz.wait()
    dtx.wait()

    @pl.when(wid == 0)
    def _tail():
        compute(zbt, xbt, ob0, _TAILR)
        pltpu.sync_copy(ob0.at[pl.ds(0, _TAILR)],
                        out_hbm.at[pl.ds(_TROW, _TAILR)])


def kernel(inputs, z, shift_table):
    n = inputs.shape[0]
    x3 = inputs.reshape(_R, 1, 128)
    zi = z.astype(jnp.int32)
    tbl = shift_table.reshape(-1)
    mesh = plsc.VectorSubcoreMesh(core_axis_name="c", subcore_axis_name="s")
    out = pl.kernel(
        _sc_body,
        out_type=jax.ShapeDtypeStruct((_R, 1, 128), jnp.float32),
        mesh=mesh,
        compiler_params=pltpu.CompilerParams(needs_layout_passes=False),
        scratch_types=[
            pltpu.VMEM((_TBL,), jnp.float32),
            pltpu.VMEM((_RC * 128,), jnp.int32),
            pltpu.VMEM((_RC * 128,), jnp.int32),
            pltpu.VMEM((_RC, 1, 128), jnp.float32),
            pltpu.VMEM((_RC, 1, 128), jnp.float32),
            pltpu.VMEM((_RC, 1, 128), jnp.float32),
            pltpu.VMEM((_RC, 1, 128), jnp.float32),
            pltpu.VMEM((_TAILR * 128,), jnp.int32),
            pltpu.VMEM((_TAILR, 1, 128), jnp.float32),
            pltpu.SemaphoreType.DMA,
            pltpu.SemaphoreType.DMA,
            pltpu.SemaphoreType.DMA,
            pltpu.SemaphoreType.DMA,
            pltpu.SemaphoreType.DMA,
            pltpu.SemaphoreType.DMA,
            pltpu.SemaphoreType.DMA,
            pltpu.SemaphoreType.DMA,
            pltpu.SemaphoreType.DMA,
        ],
    )(x3, zi, tbl)
    return out.reshape(n, 1)
